# baseline (speedup 1.0000x reference)
import functools

import jax
import jax.numpy as jnp
from jax import lax
from jax.experimental import pallas as pl
from jax.experimental.pallas import tpu as pltpu
from jax.experimental.pallas import tpu_sc as plsc

_N = 10000
_E = 320000
_NCORE = 2
_NSUB = 16
_NW = _NCORE * _NSUB
_CH = 256
_NCH = 40
_EPAD = _NW * _NCH * _CH
_NPAD = 10240
_RPT = _NPAD // _NSUB

_SELU_ALPHA = 1.6732632423543772
_SELU_SCALE = 1.0507009873554805


def _selu(v):
    return _SELU_SCALE * jnp.where(v > 0, v, _SELU_ALPHA * (jnp.exp(v) - 1.0))


def _make_agg(width, grp, stage_y=True, interpret=False):
    ngr = _NCH // grp
    mesh = plsc.VectorSubcoreMesh(
        core_axis_name="c", subcore_axis_name="s",
        num_cores=_NCORE, num_subcores=_NSUB,
    )

    @functools.partial(
        pl.kernel,
        out_type=jax.ShapeDtypeStruct((_NCORE, _NPAD, width), jnp.float32),
        mesh=mesh,
        scratch_types=[
            pltpu.VMEM((_NCH, _CH), jnp.int32),
            pltpu.VMEM((_NCH, _CH), jnp.int32),
            pltpu.VMEM((2 * grp, _CH, width), jnp.float32),
            pltpu.VMEM((_RPT, width), jnp.float32),
            pltpu.VMEM_SHARED((_NPAD, width), jnp.float32),
        ] + ([pltpu.VMEM_SHARED((_NPAD, width), jnp.float32)] if stage_y
             else []) + [
            pltpu.SemaphoreType.DMA,
            pltpu.SemaphoreType.DMA,
            pltpu.SemaphoreType.DMA,
        ],
        compiler_params=pltpu.CompilerParams(use_tc_tiling_on_sc=False),
        interpret=interpret,
    )
    def agg(y_hbm, src_hbm, dst_hbm, out_hbm, idx_s, idx_d, rows, zbuf, acc,
            *rest):
        if stage_y:
            ybuf, isem, gsem, ssem = rest
        else:
            isem, gsem, ssem = rest
            ybuf = y_hbm
        cid = lax.axis_index("c")
        sid = lax.axis_index("s")
        wid = sid * _NCORE + cid
        chunk0 = wid * _NCH
        row0 = sid * _RPT

        pltpu.async_copy(src_hbm.at[pl.ds(chunk0, _NCH)], idx_s, isem)
        pltpu.async_copy(dst_hbm.at[pl.ds(chunk0, _NCH)], idx_d, isem)
        if stage_y:
            pltpu.async_copy(
                y_hbm.at[pl.ds(row0, _RPT)], ybuf.at[pl.ds(row0, _RPT)], isem
            )

        def zrow(j, carry):
            for w in range(width // 16):
                zbuf[j, pl.ds(w * 16, 16)] = jnp.zeros((16,), jnp.float32)
            return carry

        lax.fori_loop(0, _RPT, zrow, 0)
        pltpu.sync_copy(zbuf, acc.at[pl.ds(row0, _RPT)])
        pltpu.make_async_copy(src_hbm.at[pl.ds(chunk0, _NCH)], idx_s, isem).wait()
        pltpu.make_async_copy(dst_hbm.at[pl.ds(chunk0, _NCH)], idx_d, isem).wait()
        if stage_y:
            pltpu.make_async_copy(
                y_hbm.at[pl.ds(row0, _RPT)], ybuf.at[pl.ds(row0, _RPT)], isem
            ).wait()
        plsc.subcore_barrier()

        def fire_gathers(g):
            off = (g % 2) * grp
            for b in range(grp):
                pltpu.async_copy(
                    ybuf.at[idx_s.at[g * grp + b]], rows.at[off + b], gsem
                )

        def drain(sem, g):
            off = (g % 2) * grp
            for b in range(grp):
                pltpu.make_async_copy(
                    ybuf.at[idx_s.at[g * grp + b]], rows.at[off + b], sem
                ).wait()

        fire_gathers(0)
        for g in range(ngr):
            off = (g % 2) * grp
            drain(gsem, g)
            if g + 1 < ngr:
                if g >= 1:
                    drain(ssem, g - 1)
                fire_gathers(g + 1)
            for b in range(grp):
                pltpu.async_copy(
                    rows.at[off + b], acc.at[idx_d.at[g * grp + b]], ssem,
                    add=True,
                )
        drain(ssem, ngr - 2)
        drain(ssem, ngr - 1)
        plsc.subcore_barrier()
        pltpu.sync_copy(
            acc.at[pl.ds(row0, _RPT)], out_hbm.at[cid, pl.ds(row0, _RPT)]
        )

    return agg


def _make_agg_pair(grp, interpret=False):
    width = 16
    ngr = _NCH // grp
    mesh = plsc.VectorSubcoreMesh(
        core_axis_name="c", subcore_axis_name="s",
        num_cores=_NCORE, num_subcores=_NSUB,
    )

    @functools.partial(
        pl.kernel,
        out_type=jax.ShapeDtypeStruct((2, _NCORE, _NPAD, width), jnp.float32),
        mesh=mesh,
        scratch_types=[
            pltpu.VMEM((_NCH, _CH), jnp.int32),
            pltpu.VMEM((_NCH, _CH), jnp.int32),
            pltpu.VMEM((2 * grp, _CH, width), jnp.float32),
            pltpu.VMEM((_RPT, width), jnp.float32),
            pltpu.VMEM_SHARED((_NPAD, width), jnp.float32),
            pltpu.VMEM_SHARED((_NPAD, width), jnp.float32),
            pltpu.SemaphoreType.DMA,
            pltpu.SemaphoreType.DMA,
            pltpu.SemaphoreType.DMA,
        ],
        compiler_params=pltpu.CompilerParams(use_tc_tiling_on_sc=False),
        interpret=interpret,
    )
    def agg2(yl_hbm, yr_hbm, src_hbm, dst_hbm, out_hbm, idx_s, idx_d, rows,
             zbuf, acc, ybuf, isem, gsem, ssem):
        cid = lax.axis_index("c")
        sid = lax.axis_index("s")
        wid = sid * _NCORE + cid
        chunk0 = wid * _NCH
        row0 = sid * _RPT

        pltpu.async_copy(src_hbm.at[pl.ds(chunk0, _NCH)], idx_s, isem)
        pltpu.async_copy(dst_hbm.at[pl.ds(chunk0, _NCH)], idx_d, isem)

        def zrow(j, carry):
            zbuf[j, :] = jnp.zeros((16,), jnp.float32)
            return carry

        lax.fori_loop(0, _RPT, zrow, 0)
        pltpu.make_async_copy(src_hbm.at[pl.ds(chunk0, _NCH)], idx_s, isem).wait()
        pltpu.make_async_copy(dst_hbm.at[pl.ds(chunk0, _NCH)], idx_d, isem).wait()

        def fire_gathers(g):
            off = (g % 2) * grp
            for b in range(grp):
                pltpu.async_copy(
                    ybuf.at[idx_s.at[g * grp + b]], rows.at[off + b], gsem
                )

        def drain(sem, g):
            off = (g % 2) * grp
            for b in range(grp):
                pltpu.make_async_copy(
                    ybuf.at[idx_s.at[g * grp + b]], rows.at[off + b], sem
                ).wait()

        for half, y_hbm in ((0, yl_hbm), (1, yr_hbm)):
            pltpu.sync_copy(
                y_hbm.at[pl.ds(row0, _RPT)], ybuf.at[pl.ds(row0, _RPT)]
            )
            pltpu.sync_copy(zbuf, acc.at[pl.ds(row0, _RPT)])
            plsc.subcore_barrier()
            fire_gathers(0)
            for g in range(ngr):
                off = (g % 2) * grp
                drain(gsem, g)
                if g + 1 < ngr:
                    if g >= 1:
                        drain(ssem, g - 1)
                    fire_gathers(g + 1)
                for b in range(grp):
                    pltpu.async_copy(
                        rows.at[off + b], acc.at[idx_d.at[g * grp + b]], ssem,
                        add=True,
                    )
            drain(ssem, ngr - 2)
            drain(ssem, ngr - 1)
            plsc.subcore_barrier()
            pltpu.sync_copy(
                acc.at[pl.ds(row0, _RPT)],
                out_hbm.at[half, cid, pl.ds(row0, _RPT)],
            )

    return agg2


def _proj0_body(x_ref, w_ref, y_ref):
    y_ref[pl.ds(0, _N)] = jnp.dot(
        x_ref[...], w_ref[...], preferred_element_type=jnp.float32
    )
    y_ref[pl.ds(_N, _NPAD - _N)] = jnp.zeros((_NPAD - _N, 16), jnp.float32)


def _node0_body(a_ref, y_ref, b0a_ref, w0b_ref, b0b_ref, wm1_ref, x_ref, yn_ref):
    pre = a_ref[0, :_N] + a_ref[1, :_N] + y_ref[:_N] + b0a_ref[...]
    x1 = (
        jnp.dot(_selu(pre), w0b_ref[...], preferred_element_type=jnp.float32)
        + b0b_ref[...]
    )
    x_ref[...] = x1
    yn_ref[pl.ds(0, _N)] = jnp.dot(
        x1, wm1_ref[...], preferred_element_type=jnp.float32
    )
    yn_ref[pl.ds(_N, _NPAD - _N)] = jnp.zeros((_NPAD - _N, 16), jnp.float32)


def _node_mid_body(
    x_ref, a_ref, y_ref, b1_ref, w2_ref, b2_ref, wn_ref, xo_ref, yn_ref
):
    pre = a_ref[0, :_N] + a_ref[1, :_N] + y_ref[:_N] + b1_ref[...]
    h = (
        jnp.dot(_selu(pre), w2_ref[...], preferred_element_type=jnp.float32)
        + b2_ref[...]
    )
    xn = x_ref[...] + h
    xo_ref[...] = xn
    yn_ref[pl.ds(0, _N)] = jnp.dot(
        xn, wn_ref[...], preferred_element_type=jnp.float32
    )
    yn_ref[pl.ds(_N, _NPAD - _N)] = jnp.zeros((_NPAD - _N, 16), jnp.float32)


def _node_last_body(x_ref, a_ref, y_ref, b1_ref, w2_ref, b2_ref, xo_ref,
                    xl_ref, xr_ref):
    pre = a_ref[0, :_N] + a_ref[1, :_N] + y_ref[:_N] + b1_ref[...]
    h = (
        jnp.dot(_selu(pre), w2_ref[...], preferred_element_type=jnp.float32)
        + b2_ref[...]
    )
    xn = x_ref[...] + h
    xo_ref[...] = xn
    zpad = jnp.zeros((_NPAD - _N, 16), jnp.float32)
    xl_ref[pl.ds(0, _N)] = xn[:, :16]
    xl_ref[pl.ds(_N, _NPAD - _N)] = zpad
    xr_ref[pl.ds(0, _N)] = xn[:, 16:]
    xr_ref[pl.ds(_N, _NPAD - _N)] = zpad


def _final_body(x_ref, a_ref, wl_ref, o_ref):
    zl = a_ref[0, 0, :_N] + a_ref[0, 1, :_N] + x_ref[:, :16]
    zr = a_ref[1, 0, :_N] + a_ref[1, 1, :_N] + x_ref[:, 16:]
    z = jnp.concatenate([zl, zr], axis=1)
    o_ref[...] = jnp.dot(z, wl_ref[...], preferred_element_type=jnp.float32)


def _tc(body, out_shape, *args, interpret=False):
    return pl.pallas_call(body, out_shape=out_shape, interpret=interpret)(*args)


def _gin(x, edge_index, W0a, b0a, W0b, b0b, Wm1, bm1, Wm2, bm2, Wlast,
         interpret=False):
    pad = _EPAD - _E
    src = jnp.concatenate(
        [edge_index[0], jnp.zeros((pad,), jnp.int32)]
    ).reshape(_NW * _NCH, _CH)
    dst = jnp.concatenate(
        [edge_index[1], jnp.full((pad,), _N, jnp.int32)]
    ).reshape(_NW * _NCH, _CH)
    agg16 = _make_agg(16, 8, interpret=interpret)
    agg_fin = _make_agg_pair(8, interpret=interpret)

    f32 = jnp.float32
    y = _tc(_proj0_body, jax.ShapeDtypeStruct((_NPAD, 16), f32), x, W0a,
            interpret=interpret)
    a = agg16(y, src, dst)
    xc, y = _tc(
        _node0_body,
        (jax.ShapeDtypeStruct((_N, 32), f32),
         jax.ShapeDtypeStruct((_NPAD, 16), f32)),
        a, y, b0a, W0b, b0b, Wm1[0],
        interpret=interpret,
    )
    for m in range(8):
        a = agg16(y, src, dst)
        if m < 7:
            xc, y = _tc(
                _node_mid_body,
                (jax.ShapeDtypeStruct((_N, 32), f32),
                 jax.ShapeDtypeStruct((_NPAD, 16), f32)),
                xc, a, y, bm1[m], Wm2[m], bm2[m], Wm1[m + 1],
                interpret=interpret,
            )
        else:
            xc, xl, xr = _tc(
                _node_last_body,
                (jax.ShapeDtypeStruct((_N, 32), f32),
                 jax.ShapeDtypeStruct((_NPAD, 16), f32),
                 jax.ShapeDtypeStruct((_NPAD, 16), f32)),
                xc, a, y, bm1[m], Wm2[m], bm2[m],
                interpret=interpret,
            )
    a9 = agg_fin(xl, xr, src, dst)
    out = _tc(_final_body, jax.ShapeDtypeStruct((_N, 128), f32), xc, a9, Wlast,
              interpret=interpret)
    return out


def kernel(x, edge_index, W0a, b0a, W0b, b0b, Wm1, bm1, Wm2, bm2, Wlast):
    return _gin(x, edge_index, W0a, b0a, W0b, b0b, Wm1, bm1, Wm2, bm2, Wlast)


# speedup vs baseline: 1.0278x; 1.0278x over previous
import functools

import jax
import jax.numpy as jnp
from jax import lax
from jax.experimental import pallas as pl
from jax.experimental.pallas import tpu as pltpu
from jax.experimental.pallas import tpu_sc as plsc

_N = 10000
_E = 320000
_NCORE = 2
_NSUB = 16
_NW = _NCORE * _NSUB
_CH = 512
_NCH = 20
_EPAD = _NW * _NCH * _CH
_NPAD = 10240
_RPT = _NPAD // _NSUB

_SELU_ALPHA = 1.6732632423543772
_SELU_SCALE = 1.0507009873554805


def _selu(v):
    return _SELU_SCALE * jnp.where(v > 0, v, _SELU_ALPHA * (jnp.exp(v) - 1.0))


def _make_agg(width, grp, stage_y=True, interpret=False):
    ngr = _NCH // grp
    mesh = plsc.VectorSubcoreMesh(
        core_axis_name="c", subcore_axis_name="s",
        num_cores=_NCORE, num_subcores=_NSUB,
    )

    @functools.partial(
        pl.kernel,
        out_type=jax.ShapeDtypeStruct((_NCORE, _NPAD, width), jnp.float32),
        mesh=mesh,
        scratch_types=[
            pltpu.VMEM((_NCH, _CH), jnp.int32),
            pltpu.VMEM((_NCH, _CH), jnp.int32),
            pltpu.VMEM((2 * grp, _CH, width), jnp.float32),
            pltpu.VMEM((_RPT, width), jnp.float32),
            pltpu.VMEM_SHARED((_NPAD, width), jnp.float32),
        ] + ([pltpu.VMEM_SHARED((_NPAD, width), jnp.float32)] if stage_y
             else []) + [
            pltpu.SemaphoreType.DMA,
            pltpu.SemaphoreType.DMA,
            pltpu.SemaphoreType.DMA,
        ],
        compiler_params=pltpu.CompilerParams(use_tc_tiling_on_sc=False),
        interpret=interpret,
    )
    def agg(y_hbm, src_hbm, dst_hbm, out_hbm, idx_s, idx_d, rows, zbuf, acc,
            *rest):
        if stage_y:
            ybuf, isem, gsem, ssem = rest
        else:
            isem, gsem, ssem = rest
            ybuf = y_hbm
        cid = lax.axis_index("c")
        sid = lax.axis_index("s")
        wid = sid * _NCORE + cid
        chunk0 = wid * _NCH
        row0 = sid * _RPT

        pltpu.async_copy(src_hbm.at[pl.ds(chunk0, _NCH)], idx_s, isem)
        pltpu.async_copy(dst_hbm.at[pl.ds(chunk0, _NCH)], idx_d, isem)
        if stage_y:
            pltpu.async_copy(
                y_hbm.at[pl.ds(row0, _RPT)], ybuf.at[pl.ds(row0, _RPT)], isem
            )

        def zrow(j, carry):
            for w in range(width // 16):
                zbuf[j, pl.ds(w * 16, 16)] = jnp.zeros((16,), jnp.float32)
            return carry

        lax.fori_loop(0, _RPT, zrow, 0, unroll=8)
        pltpu.sync_copy(zbuf, acc.at[pl.ds(row0, _RPT)])
        pltpu.make_async_copy(src_hbm.at[pl.ds(chunk0, _NCH)], idx_s, isem).wait()
        pltpu.make_async_copy(dst_hbm.at[pl.ds(chunk0, _NCH)], idx_d, isem).wait()
        if stage_y:
            pltpu.make_async_copy(
                y_hbm.at[pl.ds(row0, _RPT)], ybuf.at[pl.ds(row0, _RPT)], isem
            ).wait()
        plsc.subcore_barrier()

        def fire_gathers(g):
            off = (g % 2) * grp
            for b in range(grp):
                pltpu.async_copy(
                    ybuf.at[idx_s.at[g * grp + b]], rows.at[off + b], gsem
                )

        def drain(sem, g):
            off = (g % 2) * grp
            for b in range(grp):
                pltpu.make_async_copy(
                    ybuf.at[idx_s.at[g * grp + b]], rows.at[off + b], sem
                ).wait()

        fire_gathers(0)
        for g in range(ngr):
            off = (g % 2) * grp
            drain(gsem, g)
            if g + 1 < ngr:
                if g >= 1:
                    drain(ssem, g - 1)
                fire_gathers(g + 1)
            for b in range(grp):
                pltpu.async_copy(
                    rows.at[off + b], acc.at[idx_d.at[g * grp + b]], ssem,
                    add=True,
                )
        drain(ssem, ngr - 2)
        drain(ssem, ngr - 1)
        plsc.subcore_barrier()
        pltpu.sync_copy(
            acc.at[pl.ds(row0, _RPT)], out_hbm.at[cid, pl.ds(row0, _RPT)]
        )

    return agg


def _make_agg_pair(grp, interpret=False):
    width = 16
    ngr = _NCH // grp
    mesh = plsc.VectorSubcoreMesh(
        core_axis_name="c", subcore_axis_name="s",
        num_cores=_NCORE, num_subcores=_NSUB,
    )

    @functools.partial(
        pl.kernel,
        out_type=jax.ShapeDtypeStruct((2, _NCORE, _NPAD, width), jnp.float32),
        mesh=mesh,
        scratch_types=[
            pltpu.VMEM((_NCH, _CH), jnp.int32),
            pltpu.VMEM((_NCH, _CH), jnp.int32),
            pltpu.VMEM((2 * grp, _CH, width), jnp.float32),
            pltpu.VMEM((_RPT, width), jnp.float32),
            pltpu.VMEM_SHARED((_NPAD, width), jnp.float32),
            pltpu.VMEM_SHARED((_NPAD, width), jnp.float32),
            pltpu.SemaphoreType.DMA,
            pltpu.SemaphoreType.DMA,
            pltpu.SemaphoreType.DMA,
        ],
        compiler_params=pltpu.CompilerParams(use_tc_tiling_on_sc=False),
        interpret=interpret,
    )
    def agg2(yl_hbm, yr_hbm, src_hbm, dst_hbm, out_hbm, idx_s, idx_d, rows,
             zbuf, acc, ybuf, isem, gsem, ssem):
        cid = lax.axis_index("c")
        sid = lax.axis_index("s")
        wid = sid * _NCORE + cid
        chunk0 = wid * _NCH
        row0 = sid * _RPT

        pltpu.async_copy(src_hbm.at[pl.ds(chunk0, _NCH)], idx_s, isem)
        pltpu.async_copy(dst_hbm.at[pl.ds(chunk0, _NCH)], idx_d, isem)

        def zrow(j, carry):
            zbuf[j, :] = jnp.zeros((16,), jnp.float32)
            return carry

        lax.fori_loop(0, _RPT, zrow, 0, unroll=8)
        pltpu.make_async_copy(src_hbm.at[pl.ds(chunk0, _NCH)], idx_s, isem).wait()
        pltpu.make_async_copy(dst_hbm.at[pl.ds(chunk0, _NCH)], idx_d, isem).wait()

        def fire_gathers(g):
            off = (g % 2) * grp
            for b in range(grp):
                pltpu.async_copy(
                    ybuf.at[idx_s.at[g * grp + b]], rows.at[off + b], gsem
                )

        def drain(sem, g):
            off = (g % 2) * grp
            for b in range(grp):
                pltpu.make_async_copy(
                    ybuf.at[idx_s.at[g * grp + b]], rows.at[off + b], sem
                ).wait()

        for half, y_hbm in ((0, yl_hbm), (1, yr_hbm)):
            pltpu.sync_copy(
                y_hbm.at[pl.ds(row0, _RPT)], ybuf.at[pl.ds(row0, _RPT)]
            )
            pltpu.sync_copy(zbuf, acc.at[pl.ds(row0, _RPT)])
            plsc.subcore_barrier()
            fire_gathers(0)
            for g in range(ngr):
                off = (g % 2) * grp
                drain(gsem, g)
                if g + 1 < ngr:
                    if g >= 1:
                        drain(ssem, g - 1)
                    fire_gathers(g + 1)
                for b in range(grp):
                    pltpu.async_copy(
                        rows.at[off + b], acc.at[idx_d.at[g * grp + b]], ssem,
                        add=True,
                    )
            drain(ssem, ngr - 2)
            drain(ssem, ngr - 1)
            plsc.subcore_barrier()
            pltpu.sync_copy(
                acc.at[pl.ds(row0, _RPT)],
                out_hbm.at[half, cid, pl.ds(row0, _RPT)],
            )

    return agg2


def _proj0_body(x_ref, w_ref, y_ref):
    y_ref[pl.ds(0, _N)] = jnp.dot(
        x_ref[...], w_ref[...], preferred_element_type=jnp.float32
    )
    y_ref[pl.ds(_N, _NPAD - _N)] = jnp.zeros((_NPAD - _N, 16), jnp.float32)


def _node0_body(a_ref, y_ref, b0a_ref, w0b_ref, b0b_ref, wm1_ref, x_ref, yn_ref):
    pre = a_ref[0, :_N] + a_ref[1, :_N] + y_ref[:_N] + b0a_ref[...]
    x1 = (
        jnp.dot(_selu(pre), w0b_ref[...], preferred_element_type=jnp.float32)
        + b0b_ref[...]
    )
    x_ref[...] = x1
    yn_ref[pl.ds(0, _N)] = jnp.dot(
        x1, wm1_ref[...], preferred_element_type=jnp.float32
    )
    yn_ref[pl.ds(_N, _NPAD - _N)] = jnp.zeros((_NPAD - _N, 16), jnp.float32)


def _node_mid_body(
    x_ref, a_ref, y_ref, b1_ref, w2_ref, b2_ref, wn_ref, xo_ref, yn_ref
):
    pre = a_ref[0, :_N] + a_ref[1, :_N] + y_ref[:_N] + b1_ref[...]
    h = (
        jnp.dot(_selu(pre), w2_ref[...], preferred_element_type=jnp.float32)
        + b2_ref[...]
    )
    xn = x_ref[...] + h
    xo_ref[...] = xn
    yn_ref[pl.ds(0, _N)] = jnp.dot(
        xn, wn_ref[...], preferred_element_type=jnp.float32
    )
    yn_ref[pl.ds(_N, _NPAD - _N)] = jnp.zeros((_NPAD - _N, 16), jnp.float32)


def _node_last_body(x_ref, a_ref, y_ref, b1_ref, w2_ref, b2_ref, xo_ref,
                    xl_ref, xr_ref):
    pre = a_ref[0, :_N] + a_ref[1, :_N] + y_ref[:_N] + b1_ref[...]
    h = (
        jnp.dot(_selu(pre), w2_ref[...], preferred_element_type=jnp.float32)
        + b2_ref[...]
    )
    xn = x_ref[...] + h
    xo_ref[...] = xn
    zpad = jnp.zeros((_NPAD - _N, 16), jnp.float32)
    xl_ref[pl.ds(0, _N)] = xn[:, :16]
    xl_ref[pl.ds(_N, _NPAD - _N)] = zpad
    xr_ref[pl.ds(0, _N)] = xn[:, 16:]
    xr_ref[pl.ds(_N, _NPAD - _N)] = zpad


def _final_body(x_ref, a_ref, wl_ref, o_ref):
    zl = a_ref[0, 0, :_N] + a_ref[0, 1, :_N] + x_ref[:, :16]
    zr = a_ref[1, 0, :_N] + a_ref[1, 1, :_N] + x_ref[:, 16:]
    z = jnp.concatenate([zl, zr], axis=1)
    o_ref[...] = jnp.dot(z, wl_ref[...], preferred_element_type=jnp.float32)


def _tc(body, out_shape, *args, interpret=False):
    return pl.pallas_call(body, out_shape=out_shape, interpret=interpret)(*args)


def _gin(x, edge_index, W0a, b0a, W0b, b0b, Wm1, bm1, Wm2, bm2, Wlast,
         interpret=False):
    pad = _EPAD - _E
    src = jnp.concatenate(
        [edge_index[0], jnp.zeros((pad,), jnp.int32)]
    ).reshape(_NW * _NCH, _CH)
    dst = jnp.concatenate(
        [edge_index[1], jnp.full((pad,), _N, jnp.int32)]
    ).reshape(_NW * _NCH, _CH)
    agg16 = _make_agg(16, 4, interpret=interpret)
    agg_fin = _make_agg_pair(4, interpret=interpret)

    f32 = jnp.float32
    y = _tc(_proj0_body, jax.ShapeDtypeStruct((_NPAD, 16), f32), x, W0a,
            interpret=interpret)
    a = agg16(y, src, dst)
    xc, y = _tc(
        _node0_body,
        (jax.ShapeDtypeStruct((_N, 32), f32),
         jax.ShapeDtypeStruct((_NPAD, 16), f32)),
        a, y, b0a, W0b, b0b, Wm1[0],
        interpret=interpret,
    )
    for m in range(8):
        a = agg16(y, src, dst)
        if m < 7:
            xc, y = _tc(
                _node_mid_body,
                (jax.ShapeDtypeStruct((_N, 32), f32),
                 jax.ShapeDtypeStruct((_NPAD, 16), f32)),
                xc, a, y, bm1[m], Wm2[m], bm2[m], Wm1[m + 1],
                interpret=interpret,
            )
        else:
            xc, xl, xr = _tc(
                _node_last_body,
                (jax.ShapeDtypeStruct((_N, 32), f32),
                 jax.ShapeDtypeStruct((_NPAD, 16), f32),
                 jax.ShapeDtypeStruct((_NPAD, 16), f32)),
                xc, a, y, bm1[m], Wm2[m], bm2[m],
                interpret=interpret,
            )
    a9 = agg_fin(xl, xr, src, dst)
    out = _tc(_final_body, jax.ShapeDtypeStruct((_N, 128), f32), xc, a9, Wlast,
              interpret=interpret)
    return out


def kernel(x, edge_index, W0a, b0a, W0b, b0b, Wm1, bm1, Wm2, bm2, Wlast):
    return _gin(x, edge_index, W0a, b0a, W0b, b0b, Wm1, bm1, Wm2, bm2, Wlast)


# speedup vs baseline: 1.0279x; 1.0002x over previous
import functools

import jax
import jax.numpy as jnp
from jax import lax
from jax.experimental import pallas as pl
from jax.experimental.pallas import tpu as pltpu
from jax.experimental.pallas import tpu_sc as plsc

_N = 10000
_E = 320000
_NCORE = 2
_NSUB = 16
_NW = _NCORE * _NSUB
_CH = 512
_NCH = 20
_EPAD = _NW * _NCH * _CH
_NPAD = 10240
_RPT = _NPAD // _NSUB

_SELU_ALPHA = 1.6732632423543772
_SELU_SCALE = 1.0507009873554805


def _selu(v):
    return _SELU_SCALE * jnp.where(v > 0, v, _SELU_ALPHA * (jnp.exp(v) - 1.0))


def _make_agg(width, grp, stage_y=True):
    ngr = _NCH // grp
    mesh = plsc.VectorSubcoreMesh(
        core_axis_name="c", subcore_axis_name="s",
        num_cores=_NCORE, num_subcores=_NSUB,
    )

    @functools.partial(
        pl.kernel,
        out_type=jax.ShapeDtypeStruct((_NCORE, _NPAD, width), jnp.float32),
        mesh=mesh,
        scratch_types=[
            pltpu.VMEM((_NCH, _CH), jnp.int32),
            pltpu.VMEM((_NCH, _CH), jnp.int32),
            pltpu.VMEM((2 * grp, _CH, width), jnp.float32),
            pltpu.VMEM((_RPT, width), jnp.float32),
            pltpu.VMEM_SHARED((_NPAD, width), jnp.float32),
        ] + ([pltpu.VMEM_SHARED((_NPAD, width), jnp.float32)] if stage_y
             else []) + [
            pltpu.SemaphoreType.DMA,
            pltpu.SemaphoreType.DMA,
            pltpu.SemaphoreType.DMA,
        ],
        compiler_params=pltpu.CompilerParams(use_tc_tiling_on_sc=False),
    )
    def agg(y_hbm, src_hbm, dst_hbm, out_hbm, idx_s, idx_d, rows, zbuf, acc,
            *rest):
        if stage_y:
            ybuf, isem, gsem, ssem = rest
        else:
            isem, gsem, ssem = rest
            ybuf = y_hbm
        cid = lax.axis_index("c")
        sid = lax.axis_index("s")
        wid = sid * _NCORE + cid
        chunk0 = wid * _NCH
        row0 = sid * _RPT

        pltpu.async_copy(src_hbm.at[pl.ds(chunk0, _NCH)], idx_s, isem)
        pltpu.async_copy(dst_hbm.at[pl.ds(chunk0, _NCH)], idx_d, isem)
        if stage_y:
            pltpu.async_copy(
                y_hbm.at[pl.ds(row0, _RPT)], ybuf.at[pl.ds(row0, _RPT)], isem
            )

        def zrow(j, carry):
            for w in range(width // 16):
                zbuf[j, pl.ds(w * 16, 16)] = jnp.zeros((16,), jnp.float32)
            return carry

        lax.fori_loop(0, _RPT, zrow, 0, unroll=8)
        pltpu.sync_copy(zbuf, acc.at[pl.ds(row0, _RPT)])
        pltpu.make_async_copy(src_hbm.at[pl.ds(chunk0, _NCH)], idx_s, isem).wait()
        pltpu.make_async_copy(dst_hbm.at[pl.ds(chunk0, _NCH)], idx_d, isem).wait()
        if stage_y:
            pltpu.make_async_copy(
                y_hbm.at[pl.ds(row0, _RPT)], ybuf.at[pl.ds(row0, _RPT)], isem
            ).wait()
        plsc.subcore_barrier()

        def fire_gathers(g):
            off = (g % 2) * grp
            for b in range(grp):
                pltpu.async_copy(
                    ybuf.at[idx_s.at[g * grp + b]], rows.at[off + b], gsem
                )

        def drain(sem, g):
            off = (g % 2) * grp
            for b in range(grp):
                pltpu.make_async_copy(
                    ybuf.at[idx_s.at[g * grp + b]], rows.at[off + b], sem
                ).wait()

        fire_gathers(0)
        for g in range(ngr):
            off = (g % 2) * grp
            drain(gsem, g)
            if g + 1 < ngr:
                if g >= 1:
                    drain(ssem, g - 1)
                fire_gathers(g + 1)
            for b in range(grp):
                pltpu.async_copy(
                    rows.at[off + b], acc.at[idx_d.at[g * grp + b]], ssem,
                    add=True,
                )
        drain(ssem, ngr - 2)
        drain(ssem, ngr - 1)
        plsc.subcore_barrier()
        pltpu.sync_copy(
            acc.at[pl.ds(row0, _RPT)], out_hbm.at[cid, pl.ds(row0, _RPT)]
        )

    return agg


def _make_agg_pair(grp):
    width = 16
    ngr = _NCH // grp
    mesh = plsc.VectorSubcoreMesh(
        core_axis_name="c", subcore_axis_name="s",
        num_cores=_NCORE, num_subcores=_NSUB,
    )

    @functools.partial(
        pl.kernel,
        out_type=jax.ShapeDtypeStruct((2, _NCORE, _NPAD, width), jnp.float32),
        mesh=mesh,
        scratch_types=[
            pltpu.VMEM((_NCH, _CH), jnp.int32),
            pltpu.VMEM((_NCH, _CH), jnp.int32),
            pltpu.VMEM((2 * grp, _CH, width), jnp.float32),
            pltpu.VMEM((_RPT, width), jnp.float32),
            pltpu.VMEM_SHARED((_NPAD, width), jnp.float32),
            pltpu.VMEM_SHARED((_NPAD, width), jnp.float32),
            pltpu.SemaphoreType.DMA,
            pltpu.SemaphoreType.DMA,
            pltpu.SemaphoreType.DMA,
        ],
        compiler_params=pltpu.CompilerParams(use_tc_tiling_on_sc=False),
    )
    def agg2(yl_hbm, yr_hbm, src_hbm, dst_hbm, out_hbm, idx_s, idx_d, rows,
             zbuf, acc, ybuf, isem, gsem, ssem):
        cid = lax.axis_index("c")
        sid = lax.axis_index("s")
        wid = sid * _NCORE + cid
        chunk0 = wid * _NCH
        row0 = sid * _RPT

        pltpu.async_copy(src_hbm.at[pl.ds(chunk0, _NCH)], idx_s, isem)
        pltpu.async_copy(dst_hbm.at[pl.ds(chunk0, _NCH)], idx_d, isem)

        def zrow(j, carry):
            zbuf[j, :] = jnp.zeros((16,), jnp.float32)
            return carry

        lax.fori_loop(0, _RPT, zrow, 0, unroll=8)
        pltpu.make_async_copy(src_hbm.at[pl.ds(chunk0, _NCH)], idx_s, isem).wait()
        pltpu.make_async_copy(dst_hbm.at[pl.ds(chunk0, _NCH)], idx_d, isem).wait()

        def fire_gathers(g):
            off = (g % 2) * grp
            for b in range(grp):
                pltpu.async_copy(
                    ybuf.at[idx_s.at[g * grp + b]], rows.at[off + b], gsem
                )

        def drain(sem, g):
            off = (g % 2) * grp
            for b in range(grp):
                pltpu.make_async_copy(
                    ybuf.at[idx_s.at[g * grp + b]], rows.at[off + b], sem
                ).wait()

        for half, y_hbm in ((0, yl_hbm), (1, yr_hbm)):
            pltpu.sync_copy(
                y_hbm.at[pl.ds(row0, _RPT)], ybuf.at[pl.ds(row0, _RPT)]
            )
            pltpu.sync_copy(zbuf, acc.at[pl.ds(row0, _RPT)])
            plsc.subcore_barrier()
            fire_gathers(0)
            for g in range(ngr):
                off = (g % 2) * grp
                drain(gsem, g)
                if g + 1 < ngr:
                    if g >= 1:
                        drain(ssem, g - 1)
                    fire_gathers(g + 1)
                for b in range(grp):
                    pltpu.async_copy(
                        rows.at[off + b], acc.at[idx_d.at[g * grp + b]], ssem,
                        add=True,
                    )
            drain(ssem, ngr - 2)
            drain(ssem, ngr - 1)
            plsc.subcore_barrier()
            pltpu.sync_copy(
                acc.at[pl.ds(row0, _RPT)],
                out_hbm.at[half, cid, pl.ds(row0, _RPT)],
            )

    return agg2


def _proj0_body(x_ref, w_ref, y_ref):
    y_ref[pl.ds(0, _N)] = jnp.dot(
        x_ref[...], w_ref[...], preferred_element_type=jnp.float32
    )
    y_ref[pl.ds(_N, _NPAD - _N)] = jnp.zeros((_NPAD - _N, 16), jnp.float32)


def _node0_body(a_ref, y_ref, b0a_ref, w0b_ref, b0b_ref, wm1_ref, x_ref, yn_ref):
    pre = a_ref[0, :_N] + a_ref[1, :_N] + y_ref[:_N] + b0a_ref[...]
    x1 = (
        jnp.dot(_selu(pre), w0b_ref[...], preferred_element_type=jnp.float32)
        + b0b_ref[...]
    )
    x_ref[...] = x1
    yn_ref[pl.ds(0, _N)] = jnp.dot(
        x1, wm1_ref[...], preferred_element_type=jnp.float32
    )
    yn_ref[pl.ds(_N, _NPAD - _N)] = jnp.zeros((_NPAD - _N, 16), jnp.float32)


def _node_mid_body(
    x_ref, a_ref, y_ref, b1_ref, w2_ref, b2_ref, wn_ref, xo_ref, yn_ref
):
    pre = a_ref[0, :_N] + a_ref[1, :_N] + y_ref[:_N] + b1_ref[...]
    h = (
        jnp.dot(_selu(pre), w2_ref[...], preferred_element_type=jnp.float32)
        + b2_ref[...]
    )
    xn = x_ref[...] + h
    xo_ref[...] = xn
    yn_ref[pl.ds(0, _N)] = jnp.dot(
        xn, wn_ref[...], preferred_element_type=jnp.float32
    )
    yn_ref[pl.ds(_N, _NPAD - _N)] = jnp.zeros((_NPAD - _N, 16), jnp.float32)


def _node_last_body(x_ref, a_ref, y_ref, b1_ref, w2_ref, b2_ref, xo_ref,
                    xl_ref, xr_ref):
    pre = a_ref[0, :_N] + a_ref[1, :_N] + y_ref[:_N] + b1_ref[...]
    h = (
        jnp.dot(_selu(pre), w2_ref[...], preferred_element_type=jnp.float32)
        + b2_ref[...]
    )
    xn = x_ref[...] + h
    xo_ref[...] = xn
    zpad = jnp.zeros((_NPAD - _N, 16), jnp.float32)
    xl_ref[pl.ds(0, _N)] = xn[:, :16]
    xl_ref[pl.ds(_N, _NPAD - _N)] = zpad
    xr_ref[pl.ds(0, _N)] = xn[:, 16:]
    xr_ref[pl.ds(_N, _NPAD - _N)] = zpad


def _final_body(x_ref, a_ref, wl_ref, o_ref):
    zl = a_ref[0, 0, :_N] + a_ref[0, 1, :_N] + x_ref[:, :16]
    zr = a_ref[1, 0, :_N] + a_ref[1, 1, :_N] + x_ref[:, 16:]
    z = jnp.concatenate([zl, zr], axis=1)
    o_ref[...] = jnp.dot(z, wl_ref[...], preferred_element_type=jnp.float32)


def _tc(body, out_shape, *args):
    return pl.pallas_call(body, out_shape=out_shape)(*args)


def _gin(x, edge_index, W0a, b0a, W0b, b0b, Wm1, bm1, Wm2, bm2, Wlast):
    pad = _EPAD - _E
    src = jnp.concatenate(
        [edge_index[0], jnp.zeros((pad,), jnp.int32)]
    ).reshape(_NW * _NCH, _CH)
    dst = jnp.concatenate(
        [edge_index[1], jnp.full((pad,), _N, jnp.int32)]
    ).reshape(_NW * _NCH, _CH)
    agg16 = _make_agg(16, 4)
    agg_fin = _make_agg_pair(4)

    f32 = jnp.float32
    y = _tc(_proj0_body, jax.ShapeDtypeStruct((_NPAD, 16), f32), x, W0a)
    a = agg16(y, src, dst)
    xc, y = _tc(
        _node0_body,
        (jax.ShapeDtypeStruct((_N, 32), f32),
         jax.ShapeDtypeStruct((_NPAD, 16), f32)),
        a, y, b0a, W0b, b0b, Wm1[0],
            )
    for m in range(8):
        a = agg16(y, src, dst)
        if m < 7:
            xc, y = _tc(
                _node_mid_body,
                (jax.ShapeDtypeStruct((_N, 32), f32),
                 jax.ShapeDtypeStruct((_NPAD, 16), f32)),
                xc, a, y, bm1[m], Wm2[m], bm2[m], Wm1[m + 1],
            )
        else:
            xc, xl, xr = _tc(
                _node_last_body,
                (jax.ShapeDtypeStruct((_N, 32), f32),
                 jax.ShapeDtypeStruct((_NPAD, 16), f32),
                 jax.ShapeDtypeStruct((_NPAD, 16), f32)),
                xc, a, y, bm1[m], Wm2[m], bm2[m],
            )
    a9 = agg_fin(xl, xr, src, dst)
    out = _tc(_final_body, jax.ShapeDtypeStruct((_N, 128), f32),
              xc, a9, Wlast)
    return out


def kernel(x, edge_index, W0a, b0a, W0b, b0b, Wm1, bm1, Wm2, bm2, Wlast):
    return _gin(x, edge_index, W0a, b0a, W0b, b0b, Wm1, bm1, Wm2, bm2, Wlast)
